# baseline (speedup 1.0000x reference)
import functools

import jax
import jax.numpy as jnp
from jax import lax
from jax.experimental import pallas as pl
from jax.experimental.pallas import tpu as pltpu
from jax.experimental.pallas import tpu_sc as plsc

_INTERPRET = False
_USE_SC = True

E = 8
D = 1024
F = 1024
N = 2048
BM = 128
NBLK = 40
P = NBLK * BM
NC, NS = 2, 16
NW = NC * NS


def _router_body(x_ref, gw_ref, lb_ref, i1_ref, i2_ref, w1_ref, w2_ref):
    xb = x_ref[...]
    logits = lax.dot_general(xb, gw_ref[...], (((1,), (1,)), ((), ())),
                             preferred_element_type=jnp.float32)
    sel = logits + lb_ref[...]
    iota = lax.broadcasted_iota(jnp.int32, sel.shape, 1)
    neg = jnp.float32(-1e30)

    m1 = jnp.max(sel, axis=1, keepdims=True)
    idx1 = jnp.min(jnp.where(sel >= m1, iota, E), axis=1, keepdims=True)
    pick1 = iota == idx1
    s1 = jnp.sum(jnp.where(pick1, logits, 0.0), axis=1, keepdims=True)

    sel2 = jnp.where(pick1, neg, sel)
    m2 = jnp.max(sel2, axis=1, keepdims=True)
    idx2 = jnp.min(jnp.where(sel2 >= m2, iota, E), axis=1, keepdims=True)
    pick2 = iota == idx2
    s2 = jnp.sum(jnp.where(pick2, logits, 0.0), axis=1, keepdims=True)

    g1 = 1.0 / (1.0 + jnp.exp(-s1))
    g2 = 1.0 / (1.0 + jnp.exp(-s2))
    denom = g1 + g2 + 1e-6
    i1_ref[...] = idx1
    i2_ref[...] = idx2
    w1_ref[...] = g1 / denom
    w2_ref[...] = g2 / denom


def _router(x2d, gate_w, lb2d):
    full = lambda: pl.BlockSpec((N, 1), lambda: (0, 0))
    return pl.pallas_call(
        _router_body,
        in_specs=[
            pl.BlockSpec((N, D), lambda: (0, 0)),
            pl.BlockSpec((E, D), lambda: (0, 0)),
            pl.BlockSpec((1, E), lambda: (0, 0)),
        ],
        out_specs=[full(), full(), full(), full()],
        out_shape=[
            jax.ShapeDtypeStruct((N, 1), jnp.int32),
            jax.ShapeDtypeStruct((N, 1), jnp.int32),
            jax.ShapeDtypeStruct((N, 1), jnp.float32),
            jax.ShapeDtypeStruct((N, 1), jnp.float32),
        ],
        interpret=_INTERPRET,
    )(x2d, gate_w, lb2d)


def _metadata(i1, i2):
    flat_e = jnp.concatenate([i1, i2], axis=1).reshape(-1)
    oh = (flat_e[:, None] == jnp.arange(E, dtype=jnp.int32)[None, :]).astype(jnp.int32)
    cum = jnp.cumsum(oh, axis=0)
    counts = cum[-1]
    nblk_e = (counts + BM - 1) // BM
    blk_end = jnp.cumsum(nblk_e)
    blk_start = blk_end - nblk_e
    base_e = blk_start * BM
    rank = jnp.sum(cum * oh, axis=1) - 1
    pos = rank + jnp.take(base_e, flat_e)
    tok = jnp.arange(2 * N, dtype=jnp.int32) // 2
    slot_token = jnp.zeros((P,), jnp.int32).at[pos].set(tok)
    g_ids = jnp.arange(NBLK, dtype=jnp.int32)
    block_expert = jnp.minimum(
        jnp.sum((g_ids[:, None] >= blk_end[None, :]).astype(jnp.int32), axis=1), E - 1)
    pos2 = pos.reshape(N, 2)
    return slot_token, block_expert, pos2[:, 0], pos2[:, 1]


def _sc_gather(table, idx, n_rows):
    rows_per_w = n_rows // NW
    ch = rows_per_w
    while ch * D * 4 > 220 * 1024:
        ch //= 2
    n_ch = rows_per_w // ch
    mesh = plsc.VectorSubcoreMesh(core_axis_name="c", subcore_axis_name="s",
                                  num_cores=NC, num_subcores=NS)

    @functools.partial(
        pl.kernel,
        out_type=jax.ShapeDtypeStruct((n_rows, D), jnp.float32),
        mesh=mesh,
        scratch_types=[
            pltpu.VMEM((rows_per_w,), jnp.int32),
            pltpu.VMEM((ch, D), jnp.float32),
            pltpu.VMEM((ch, D), jnp.float32),
            pltpu.SemaphoreType.DMA,
            pltpu.SemaphoreType.DMA,
        ],
    )
    def k(table_hbm, idx_hbm, out_hbm, idx_v, buf0, buf1, sem0, sem1):
        wid = lax.axis_index("s") * NC + lax.axis_index("c")
        base = wid * rows_per_w
        pltpu.sync_copy(idx_hbm.at[pl.ds(base, rows_per_w)], idx_v)
        bufs = (buf0, buf1)
        sems = (sem0, sem1)
        descs = [None, None]
        for c in range(n_ch):
            descs[c % 2] = pltpu.async_copy(
                table_hbm.at[idx_v.at[pl.ds(c * ch, ch)]], bufs[c % 2], sems[c % 2])
            if c > 0:
                descs[(c - 1) % 2].wait()
                pltpu.sync_copy(bufs[(c - 1) % 2],
                                out_hbm.at[pl.ds(base + (c - 1) * ch, ch)])
        descs[(n_ch - 1) % 2].wait()
        pltpu.sync_copy(bufs[(n_ch - 1) % 2],
                        out_hbm.at[pl.ds(base + (n_ch - 1) * ch, ch)])

    return k(table, idx)


def _ffn_body(be_ref, xg_ref, w1_ref, w2_ref, y_ref):
    xb16 = xg_ref[...].astype(jnp.bfloat16)
    h = lax.dot_general(xb16, w1_ref[0], (((1,), (1,)), ((), ())),
                        preferred_element_type=jnp.float32)
    h = jnp.square(jnp.maximum(h, 0.0))
    y_ref[...] = lax.dot_general(h.astype(jnp.bfloat16), w2_ref[0],
                                 (((1,), (1,)), ((), ())),
                                 preferred_element_type=jnp.float32)


def _ffn(xg, w1_16, w2_16, block_expert):
    grid_spec = pltpu.PrefetchScalarGridSpec(
        num_scalar_prefetch=1,
        grid=(NBLK,),
        in_specs=[
            pl.BlockSpec((BM, D), lambda g, be: (g, 0)),
            pl.BlockSpec((1, F, D), lambda g, be: (be[g], 0, 0)),
            pl.BlockSpec((1, D, F), lambda g, be: (be[g], 0, 0)),
        ],
        out_specs=pl.BlockSpec((BM, D), lambda g, be: (g, 0)),
    )
    return pl.pallas_call(
        _ffn_body,
        grid_spec=grid_spec,
        out_shape=jax.ShapeDtypeStruct((P, D), jnp.float32),
        interpret=_INTERPRET,
    )(block_expert, xg, w1_16, w2_16)


def _sc_fetch2(y, pos0, pos1):
    rows_per_w = N // NW
    ch = 32
    n_ch = rows_per_w // ch
    mesh = plsc.VectorSubcoreMesh(core_axis_name="c", subcore_axis_name="s",
                                  num_cores=NC, num_subcores=NS)

    @functools.partial(
        pl.kernel,
        out_type=(jax.ShapeDtypeStruct((N, D), jnp.float32),
                  jax.ShapeDtypeStruct((N, D), jnp.float32)),
        mesh=mesh,
        scratch_types=[
            pltpu.VMEM((rows_per_w,), jnp.int32),
            pltpu.VMEM((rows_per_w,), jnp.int32),
            pltpu.VMEM((ch, D), jnp.float32),
            pltpu.VMEM((ch, D), jnp.float32),
            pltpu.SemaphoreType.DMA,
            pltpu.SemaphoreType.DMA,
        ],
    )
    def k(y_hbm, p0_hbm, p1_hbm, a_hbm, b_hbm, p0_v, p1_v, buf0, buf1, sem0, sem1):
        wid = lax.axis_index("s") * NC + lax.axis_index("c")
        base = wid * rows_per_w
        pltpu.sync_copy(p0_hbm.at[pl.ds(base, rows_per_w)], p0_v)
        pltpu.sync_copy(p1_hbm.at[pl.ds(base, rows_per_w)], p1_v)
        bufs = (buf0, buf1)
        sems = (sem0, sem1)
        work = []
        for c in range(n_ch):
            work.append((p0_v, a_hbm, c))
        for c in range(n_ch):
            work.append((p1_v, b_hbm, c))
        descs = [None, None]
        prev = [None, None]
        for i, (pv, dst, c) in enumerate(work):
            descs[i % 2] = pltpu.async_copy(
                y_hbm.at[pv.at[pl.ds(c * ch, ch)]], bufs[i % 2], sems[i % 2])
            if i > 0:
                j = (i - 1) % 2
                descs[j].wait()
                pdst, pc = prev[j]
                pltpu.sync_copy(bufs[j], pdst.at[pl.ds(base + pc * ch, ch)])
            prev[i % 2] = (dst, c)
        j = (len(work) - 1) % 2
        descs[j].wait()
        pdst, pc = prev[j]
        pltpu.sync_copy(bufs[j], pdst.at[pl.ds(base + pc * ch, ch)])

    return k(y, pos0, pos1)


_CB = 256


def _combine_body(x_ref, a_ref, b_ref, wa_ref, wb_ref, sfc_ref, spr_ref, out_ref):
    xb16 = x_ref[...].astype(jnp.bfloat16)
    hs = lax.dot_general(xb16, sfc_ref[...], (((1,), (1,)), ((), ())),
                         preferred_element_type=jnp.float32)
    hs = jnp.square(jnp.maximum(hs, 0.0))
    sh = lax.dot_general(hs.astype(jnp.bfloat16), spr_ref[...],
                         (((1,), (1,)), ((), ())),
                         preferred_element_type=jnp.float32)
    out_ref[...] = sh + wa_ref[...] * a_ref[...] + wb_ref[...] * b_ref[...]


def _combine(x2d, a, b, wa, wb, sfc16, spr16):
    blk = lambda g: (g, 0)
    return pl.pallas_call(
        _combine_body,
        grid=(N // _CB,),
        in_specs=[
            pl.BlockSpec((_CB, D), blk),
            pl.BlockSpec((_CB, D), blk),
            pl.BlockSpec((_CB, D), blk),
            pl.BlockSpec((_CB, 1), blk),
            pl.BlockSpec((_CB, 1), blk),
            pl.BlockSpec((F, D), lambda g: (0, 0)),
            pl.BlockSpec((D, F), lambda g: (0, 0)),
        ],
        out_specs=pl.BlockSpec((_CB, D), blk),
        out_shape=jax.ShapeDtypeStruct((N, D), jnp.float32),
        interpret=_INTERPRET,
    )(x2d, a, b, wa, wb, sfc16, spr16)


def kernel(x, gate_w, lb_bias, w1, w2, shared_fc, shared_proj):
    bsz, t, d = x.shape
    x2d = x.reshape(t * bsz, d)
    i1, i2, wa, wb = _router(x2d, gate_w, lb_bias.reshape(1, E))
    slot_token, block_expert, pos0, pos1 = _metadata(i1, i2)
    if _USE_SC:
        xg = _sc_gather(x2d, slot_token, P)
    else:
        xg = jnp.take(x2d, slot_token, axis=0)
    y = _ffn(xg, w1.astype(jnp.bfloat16), w2.astype(jnp.bfloat16), block_expert)
    if _USE_SC:
        a, b = _sc_fetch2(y, pos0, pos1)
    else:
        a = jnp.take(y, pos0, axis=0)
        b = jnp.take(y, pos1, axis=0)
    out = _combine(x2d, a, b, wa, wb,
                   shared_fc.astype(jnp.bfloat16), shared_proj.astype(jnp.bfloat16))
    return out.reshape(bsz, t, d)
